# initial kernel scaffold (unmeasured)
import jax
import jax.numpy as jnp
from jax import lax
from jax.experimental import pallas as pl
from jax.experimental.pallas import tpu as pltpu

N_DEV = 4
M_PER = 2048
K = 8192
N_PER = 1024


def _all_gather_ring(x):

    def body(x_ref, gx_ref, local_sem, send_sems, recv_sems):
        my = lax.axis_index("i")
        right = lax.rem(my + 1, N_DEV)

        cp = pltpu.make_async_copy(
            x_ref, gx_ref.at[pl.ds(my * M_PER, M_PER)], local_sem
        )
        cp.start()
        cp.wait()

        for h in range(N_DEV - 1):
            origin = lax.rem(my - h + N_DEV, N_DEV)
            sl = pl.ds(origin * M_PER, M_PER)
            rdma = pltpu.make_async_remote_copy(
                src_ref=gx_ref.at[sl],
                dst_ref=gx_ref.at[sl],
                send_sem=send_sems.at[h],
                recv_sem=recv_sems.at[h],
                device_id=(right,),
                device_id_type=pl.DeviceIdType.MESH,
            )
            rdma.start()
            rdma.wait()

    return pl.pallas_call(
        body,
        out_shape=jax.ShapeDtypeStruct((N_DEV * M_PER, K), jnp.bfloat16),
        in_specs=[pl.BlockSpec(memory_space=pltpu.ANY)],
        out_specs=pl.BlockSpec(memory_space=pltpu.ANY),
        scratch_shapes=[
            pltpu.SemaphoreType.DMA,
            pltpu.SemaphoreType.DMA((N_DEV - 1,)),
            pltpu.SemaphoreType.DMA((N_DEV - 1,)),
        ],
    )(x)


def _gemm_gelu(gx, w):
    M = N_DEV * M_PER
    bm, bk, bn = 512, 2048, N_PER
    n_k = K // bk

    def body(x_ref, w_ref, o_ref, acc_ref):
        k = pl.program_id(1)

        @pl.when(k == 0)
        def _():
            acc_ref[...] = jnp.zeros_like(acc_ref)

        acc_ref[...] += jnp.dot(
            x_ref[...], w_ref[...], preferred_element_type=jnp.float32
        )

        @pl.when(k == n_k - 1)
        def _():
            o_ref[...] = jax.nn.gelu(acc_ref[...], approximate=True)

    return pl.pallas_call(
        body,
        grid=(M // bm, n_k),
        in_specs=[
            pl.BlockSpec((bm, bk), lambda m, k: (m, k)),
            pl.BlockSpec((bk, bn), lambda m, k: (k, 0)),
        ],
        out_specs=pl.BlockSpec((bm, bn), lambda m, k: (m, 0)),
        out_shape=jax.ShapeDtypeStruct((M, N_PER), jnp.float32),
        scratch_shapes=[pltpu.VMEM((bm, bn), jnp.float32)],
    )(gx, w)


def kernel(x, w_mat):
    x = x.astype(jnp.bfloat16)
    w_mat = w_mat.astype(jnp.bfloat16)
    gx = _all_gather_ring(x)
    return _gemm_gelu(gx, w_mat)


# baseline (device time: 2329740 ns/iter reference)
import jax
import jax.numpy as jnp
from jax import lax
from jax.experimental import pallas as pl
from jax.experimental.pallas import tpu as pltpu

N_DEV = 4
M_PER = 2048
K = 8192
N_PER = 1024


def _all_gather_ring(x):

    def body(x_ref, gx_ref, local_sem, send_sems, recv_sems):
        my = lax.axis_index("i")
        right = lax.rem(my + 1, N_DEV)

        cp = pltpu.make_async_copy(
            x_ref, gx_ref.at[pl.ds(my * M_PER, M_PER)], local_sem
        )
        cp.start()
        cp.wait()

        for h in range(N_DEV - 1):
            origin = lax.rem(my - h + N_DEV, N_DEV)
            sl = pl.ds(origin * M_PER, M_PER)
            rdma = pltpu.make_async_remote_copy(
                src_ref=gx_ref.at[sl],
                dst_ref=gx_ref.at[sl],
                send_sem=send_sems.at[h],
                recv_sem=recv_sems.at[h],
                device_id=(right,),
                device_id_type=pl.DeviceIdType.MESH,
            )
            rdma.start()
            rdma.wait()

    return pl.pallas_call(
        body,
        out_shape=jax.ShapeDtypeStruct((N_DEV * M_PER, K), jnp.bfloat16),
        in_specs=[pl.BlockSpec(memory_space=pl.ANY)],
        out_specs=pl.BlockSpec(memory_space=pl.ANY),
        scratch_shapes=[
            pltpu.SemaphoreType.DMA,
            pltpu.SemaphoreType.DMA((N_DEV - 1,)),
            pltpu.SemaphoreType.DMA((N_DEV - 1,)),
        ],
    )(x)


def _gemm_gelu(gx, w):
    M = N_DEV * M_PER
    bm, bk, bn = 512, 2048, N_PER
    n_k = K // bk

    def body(x_ref, w_ref, o_ref, acc_ref):
        k = pl.program_id(1)

        @pl.when(k == 0)
        def _():
            acc_ref[...] = jnp.zeros_like(acc_ref)

        acc_ref[...] += jnp.dot(
            x_ref[...], w_ref[...], preferred_element_type=jnp.float32
        )

        @pl.when(k == n_k - 1)
        def _():
            o_ref[...] = jax.nn.gelu(acc_ref[...], approximate=True)

    return pl.pallas_call(
        body,
        grid=(M // bm, n_k),
        in_specs=[
            pl.BlockSpec((bm, bk), lambda m, k: (m, k)),
            pl.BlockSpec((bk, bn), lambda m, k: (k, 0)),
        ],
        out_specs=pl.BlockSpec((bm, bn), lambda m, k: (m, 0)),
        out_shape=jax.ShapeDtypeStruct((M, N_PER), jnp.float32),
        scratch_shapes=[pltpu.VMEM((bm, bn), jnp.float32)],
    )(gx, w)


def kernel(x, w_mat):
    x = x.astype(jnp.bfloat16)
    w_mat = w_mat.astype(jnp.bfloat16)
    gx = _all_gather_ring(x)
    return _gemm_gelu(gx, w_mat)


# device time: 1244519 ns/iter; 1.8720x vs baseline; 1.8720x over previous
import jax
import jax.numpy as jnp
from jax import lax
from jax.experimental import pallas as pl
from jax.experimental.pallas import tpu as pltpu

N_DEV = 4
M_PER = 2048
K = 8192
N_PER = 1024
HALF = M_PER // 2
R = 4
U = HALF // R
NPOS = 3 * R
NSLOT = 3


def _all_gather_bidir(x):

    def body(x_ref, gx_ref, slots, own_sem, copy_sems, send_sems, recv_sems):
        my = lax.axis_index("i")
        right = lax.rem(my + 1, N_DEV)
        left = lax.rem(my + N_DEV - 1, N_DEV)

        own = pltpu.make_async_copy(
            x_ref, gx_ref.at[pl.ds(my * M_PER, M_PER)], own_sem
        )
        own.start()

        copies = []
        for r in range(R):
            for h in range(3):
                p = r * 3 + h
                rdmas = []
                for d in range(2):
                    tgt = right if d == 0 else left
                    base = 0 if d == 0 else HALF
                    if h == 0:
                        src = x_ref.at[pl.ds(base + r * U, U)]
                    else:
                        src = slots.at[d, (p - 1) % NSLOT]
                    rd = pltpu.make_async_remote_copy(
                        src_ref=src,
                        dst_ref=slots.at[d, p % NSLOT],
                        send_sem=send_sems.at[d, p],
                        recv_sem=recv_sems.at[d, p],
                        device_id=(tgt,),
                        device_id_type=pl.DeviceIdType.MESH,
                    )
                    rd.start()
                    rdmas.append(rd)
                for rd in rdmas:
                    rd.wait()
                for d in range(2):
                    if d == 0:
                        o = lax.rem(my + N_DEV - 1 - h, N_DEV)
                    else:
                        o = lax.rem(my + 1 + h, N_DEV)
                    row = o * M_PER + (0 if d == 0 else HALF) + r * U
                    c = pltpu.make_async_copy(
                        slots.at[d, p % NSLOT],
                        gx_ref.at[pl.ds(row, U)],
                        copy_sems.at[d, p],
                    )
                    c.start()
                    copies.append(c)

        own.wait()
        for c in copies:
            c.wait()

    return pl.pallas_call(
        body,
        out_shape=jax.ShapeDtypeStruct((N_DEV * M_PER, K), jnp.bfloat16),
        in_specs=[pl.BlockSpec(memory_space=pl.ANY)],
        out_specs=pl.BlockSpec(memory_space=pl.ANY),
        scratch_shapes=[
            pltpu.VMEM((2, NSLOT, U, K), jnp.bfloat16),
            pltpu.SemaphoreType.DMA,
            pltpu.SemaphoreType.DMA((2, NPOS)),
            pltpu.SemaphoreType.DMA((2, NPOS)),
            pltpu.SemaphoreType.DMA((2, NPOS)),
        ],
    )(x)


def _gemm_gelu(gx, w):
    M = N_DEV * M_PER
    bm, bk, bn = 512, 2048, N_PER
    n_k = K // bk

    def body(x_ref, w_ref, o_ref, acc_ref):
        k = pl.program_id(1)

        @pl.when(k == 0)
        def _():
            acc_ref[...] = jnp.zeros_like(acc_ref)

        acc_ref[...] += jnp.dot(
            x_ref[...], w_ref[...], preferred_element_type=jnp.float32
        )

        @pl.when(k == n_k - 1)
        def _():
            o_ref[...] = jax.nn.gelu(acc_ref[...], approximate=True)

    return pl.pallas_call(
        body,
        grid=(M // bm, n_k),
        in_specs=[
            pl.BlockSpec((bm, bk), lambda m, k: (m, k)),
            pl.BlockSpec((bk, bn), lambda m, k: (k, 0)),
        ],
        out_specs=pl.BlockSpec((bm, bn), lambda m, k: (m, 0)),
        out_shape=jax.ShapeDtypeStruct((M, N_PER), jnp.float32),
        scratch_shapes=[pltpu.VMEM((bm, bn), jnp.float32)],
    )(gx, w)


def kernel(x, w_mat):
    x = x.astype(jnp.bfloat16)
    w_mat = w_mat.astype(jnp.bfloat16)
    gx = _all_gather_bidir(x)
    return _gemm_gelu(gx, w_mat)
